# baseline (speedup 1.0000x reference)
import functools

import jax
import jax.numpy as jnp
from jax import lax
from jax.experimental import pallas as pl
from jax.experimental.pallas import tpu as pltpu
from jax.experimental.pallas import tpu_sc as plsc

DIM = 64
NC = 2
NS = 16
NW = NC * NS
BG = 128


def _make_gather(batch, hist):
    bpt = batch // NW
    nbg = bpt // BG
    npairs = hist * nbg // 2

    mesh = plsc.VectorSubcoreMesh(core_axis_name="c", subcore_axis_name="s")

    @functools.partial(
        pl.kernel,
        mesh=mesh,
        out_type=jax.ShapeDtypeStruct((hist, DIM, batch), jnp.float32),
        scratch_types=[
            pltpu.VMEM((BG,), jnp.int32),
            pltpu.VMEM((BG,), jnp.int32),
            pltpu.VMEM((BG,), jnp.int32),
            pltpu.VMEM((BG,), jnp.int32),
            pltpu.VMEM((BG, 2 * DIM), jnp.float32),
            pltpu.VMEM((BG, 2 * DIM), jnp.float32),
            pltpu.VMEM((DIM, BG), jnp.float32),
            pltpu.VMEM((DIM, BG), jnp.float32),
            pltpu.SemaphoreType.DMA,
            pltpu.SemaphoreType.DMA,
            pltpu.SemaphoreType.DMA,
            pltpu.SemaphoreType.DMA,
        ],
        compiler_params=pltpu.CompilerParams(
            needs_layout_passes=False, disable_bounds_checks=True
        ),
    )
    def gather(xt_hbm, w2_hbm, out_hbm, ridx0, ridx1, vr0, vr1,
               gbuf0, gbuf1, slab0, slab1, g0, g1, s0, s1):
        wid = lax.axis_index("s") * NC + lax.axis_index("c")
        tb0 = wid * bpt
        lane = lax.iota(jnp.int32, 16)
        bvec = [lane + 16 * bq for bq in range(BG // 16)]

        def unit_pos(u):
            return u // nbg, tb0 + (u % nbg) * BG

        def prep(u, ridx, vr):
            h, babs = unit_pos(u)
            pltpu.sync_copy(xt_hbm.at[h, pl.ds(babs, BG)], ridx)
            for q in range(BG // 16):
                vr[pl.ds(16 * q, 16)] = ridx[pl.ds(16 * q, 16)] >> 1

        def fire(vr, gbuf, sem):
            pltpu.async_copy(w2_hbm.at[vr], gbuf, sem)

        def wait_gather(gbuf, sem):
            pltpu.make_async_copy(w2_hbm.at[pl.ds(0, BG)], gbuf, sem).wait()

        def wait_store(slab, sem):
            pltpu.make_async_copy(
                slab, out_hbm.at[0, pl.ds(0, DIM), pl.ds(0, BG)], sem
            ).wait()

        def transpose(ridx, gbuf, slab):
            pcol = []
            for bq in range(BG // 16):
                rvec = ridx[pl.ds(16 * bq, 16)]
                pcol.append((rvec & 1) * DIM)

            zero = jnp.zeros((16,), jnp.float32)

            def dstep(dq, carry):
                d0 = dq * 16
                for du in range(16):
                    for bq in range(BG // 16):
                        slab[d0 + du, pl.ds(16 * bq, 16)] = zero
                cb = [pcol[bq] + d0 for bq in range(BG // 16)]
                rotk = lane
                for k in range(16):
                    drow = rotk + d0
                    for bq in range(BG // 16):
                        vals = plsc.load_gather(
                            gbuf, [bvec[bq], cb[bq] + rotk]
                        )
                        plsc.addupdate_scatter(
                            slab, [drow, bvec[bq]], vals
                        )
                    rotk = (rotk + 1) & 15
                return carry

            lax.fori_loop(0, DIM // 16, dstep, 0)

        def store(u, slab, sem):
            h, babs = unit_pos(u)
            pltpu.async_copy(
                slab, out_hbm.at[h, pl.ds(0, DIM), pl.ds(babs, BG)], sem
            )

        def pair(p, carry):
            ua = 2 * p
            prep(ua + 1, ridx1, vr1)
            fire(vr1, gbuf1, g1)

            wait_gather(gbuf0, g0)

            @pl.when(p > 0)
            def _():
                wait_store(slab0, s0)

            transpose(ridx0, gbuf0, slab0)
            store(ua, slab0, s0)

            @pl.when(p < npairs - 1)
            def _():
                prep(ua + 2, ridx0, vr0)
                fire(vr0, gbuf0, g0)

            wait_gather(gbuf1, g1)

            @pl.when(p > 0)
            def _():
                wait_store(slab1, s1)

            transpose(ridx1, gbuf1, slab1)
            store(ua + 1, slab1, s1)
            return carry

        prep(0, ridx0, vr0)
        fire(vr0, gbuf0, g0)
        lax.fori_loop(0, npairs, pair, 0)
        wait_store(slab0, s0)
        wait_store(slab1, s1)

    return gather


def kernel(x, weight):
    batch, hist = x.shape
    xt = x.T
    w2 = weight.reshape(weight.shape[0] // 2, 2 * DIM)
    yt = _make_gather(batch, hist)(xt, w2)
    return yt.transpose(2, 0, 1)


# speedup vs baseline: 1.0598x; 1.0598x over previous
import functools

import jax
import jax.numpy as jnp
from jax import lax
from jax.experimental import pallas as pl
from jax.experimental.pallas import tpu as pltpu
from jax.experimental.pallas import tpu_sc as plsc

DIM = 64
NC = 2
NS = 16
NW = NC * NS
IPS = 128
SUB = 4
C_ROWS = SUB * IPS


def _make_gather(batch):
    bpw = batch // NW
    npairs = bpw // (2 * C_ROWS)
    rows2_pw = bpw // IPS

    mesh = plsc.VectorSubcoreMesh(core_axis_name="c", subcore_axis_name="s")

    @functools.partial(
        pl.kernel,
        mesh=mesh,
        out_type=jax.ShapeDtypeStruct((batch, DIM), jnp.float32),
        scratch_types=[
            pltpu.VMEM((rows2_pw, IPS), jnp.int32),
            pltpu.VMEM((C_ROWS, DIM), jnp.float32),
            pltpu.VMEM((C_ROWS, DIM), jnp.float32),
            pltpu.SemaphoreType.DMA,
            pltpu.SemaphoreType.DMA,
            pltpu.SemaphoreType.DMA,
            pltpu.SemaphoreType.DMA,
        ],
        compiler_params=pltpu.CompilerParams(use_tc_tiling_on_sc=False),
    )
    def gather(x2_hbm, w_hbm, out_hbm, idx_v, buf0, buf1, g0, g1, s0, s1):
        wid = lax.axis_index("s") * NC + lax.axis_index("c")
        base_row = wid * bpw

        pltpu.sync_copy(x2_hbm.at[pl.ds(wid * rows2_pw, rows2_pw)], idx_v)

        def fire(k, buf, sem):
            return [
                pltpu.async_copy(
                    w_hbm.at[idx_v.at[k * SUB + j]],
                    buf.at[pl.ds(j * IPS, IPS)],
                    sem,
                )
                for j in range(SUB)
            ]

        def out_slot(k):
            return out_hbm.at[pl.ds(base_row + k * C_ROWS, C_ROWS)]

        def wait_bytes(buf, sem):
            pltpu.make_async_copy(buf, out_slot(0), sem).wait()

        def pair(p, carry):
            ka = 2 * p
            @pl.when(p > 0)
            def _():
                wait_bytes(buf0, s0)

            ga = fire(ka, buf0, g0)

            @pl.when(p > 0)
            def _():
                wait_bytes(buf1, g1)
                pltpu.async_copy(buf1, out_slot(ka - 1), s1)
                wait_bytes(buf1, s1)

            fire(ka + 1, buf1, g1)
            for cp in ga:
                cp.wait()
            pltpu.async_copy(buf0, out_slot(ka), s0)
            return carry

        lax.fori_loop(0, npairs, pair, 0)

        wait_bytes(buf0, s0)
        wait_bytes(buf1, g1)
        pltpu.async_copy(buf1, out_slot(2 * npairs - 1), s1)
        wait_bytes(buf1, s1)

    return gather


def kernel(x, weight):
    batch = x.shape[0] * x.shape[1]
    x2 = x.reshape(batch // IPS, IPS).astype(jnp.int32)
    out = _make_gather(batch)(x2, weight)
    return out.reshape(x.shape[0], x.shape[1], DIM)
